# Initial kernel scaffold; baseline (speedup 1.0000x reference)
import jax
import jax.numpy as jnp
from jax import lax
from jax.experimental import pallas as pl
from jax.experimental.pallas import tpu as pltpu
from jax.experimental.pallas import tpu_sc as plsc

D = 64
T = 256
B = 1024
N = B * T
NC = 2
NS = 16
NW = NC * NS
BPW = N // NW
C = 512
NCHUNK = BPW // C
SREP = C // T


def _body(x_hbm, tok_hbm, pos_hbm, out_hbm, idx_v, rows_v, pos_v, sem):
    wid = lax.axis_index("s") * NC + lax.axis_index("c")
    base = wid * BPW
    pltpu.sync_copy(pos_hbm, pos_v)

    for c in range(NCHUNK):
        off = base + c * C
        pltpu.sync_copy(x_hbm.at[pl.ds(off, C)], idx_v)
        pltpu.async_copy(tok_hbm.at[idx_v], rows_v, sem).wait()

        def add_body(i, _):
            t = i >> 2
            j16 = (i & 3) * 16
            vp = pos_v[t, pl.ds(j16, 16)]
            for s in range(SREP):
                rows_v[s * T + t, pl.ds(j16, 16)] += vp
            return 0

        lax.fori_loop(0, T * (D // 16), add_body, 0)
        pltpu.sync_copy(rows_v, out_hbm.at[pl.ds(off, C)])


def kernel(x, token_table, pos_table):
    xf = x.reshape(N).astype(jnp.int32)
    run = pl.kernel(
        _body,
        out_type=jax.ShapeDtypeStruct((N, D), jnp.float32),
        mesh=plsc.VectorSubcoreMesh(core_axis_name="c", subcore_axis_name="s"),
        scratch_types=[
            pltpu.VMEM((C,), jnp.int32),
            pltpu.VMEM((C, D), jnp.float32),
            pltpu.VMEM((T, D), jnp.float32),
            pltpu.SemaphoreType.DMA,
        ],
    )
    out = run(xf, token_table, pos_table)
    return out.reshape(B, T, D)


# speedup vs baseline: 1.2821x; 1.2821x over previous
import jax
import jax.numpy as jnp
from jax import lax
from jax.experimental import pallas as pl
from jax.experimental.pallas import tpu as pltpu
from jax.experimental.pallas import tpu_sc as plsc

D = 64
T = 256
B = 1024
N = B * T
NC = 2
NS = 16
NW = NC * NS
BPW = N // NW
C = 512
NCHUNK = BPW // C
SREP = C // T


def _body(x_hbm, tok_hbm, pos_hbm, out_hbm, idx_v, rows_v, pos_v, sem):
    wid = lax.axis_index("s") * NC + lax.axis_index("c")
    base = wid * BPW
    pltpu.sync_copy(pos_hbm, pos_v)

    for c in range(NCHUNK):
        off = base + c * C
        pltpu.sync_copy(x_hbm.at[pl.ds(off, C)], idx_v)
        pltpu.async_copy(tok_hbm.at[idx_v], rows_v, sem).wait()

        def add_body(i, _):
            t = i >> 2
            j16 = (i & 3) * 16
            vp = pos_v[t, pl.ds(j16, 16)]
            for s in range(SREP):
                rows_v[s * T + t, pl.ds(j16, 16)] += vp
            return 0

        lax.fori_loop(0, T * (D // 16), add_body, 0)
        pltpu.sync_copy(rows_v, out_hbm.at[pl.ds(off, C)])


def kernel(x, token_table, pos_table):
    xf = x.reshape(N).astype(jnp.int32)
    run = pl.kernel(
        _body,
        out_type=jax.ShapeDtypeStruct((N, D), jnp.float32),
        mesh=plsc.VectorSubcoreMesh(core_axis_name="c", subcore_axis_name="s"),
        compiler_params=pltpu.CompilerParams(use_tc_tiling_on_sc=False),
        scratch_types=[
            pltpu.VMEM((C,), jnp.int32),
            pltpu.VMEM((C, D), jnp.float32),
            pltpu.VMEM((T, D), jnp.float32),
            pltpu.SemaphoreType.DMA,
        ],
    )
    out = run(xf, token_table, pos_table)
    return out.reshape(B, T, D)


# speedup vs baseline: 1.5658x; 1.2213x over previous
import jax
import jax.numpy as jnp
from jax import lax
from jax.experimental import pallas as pl
from jax.experimental.pallas import tpu as pltpu
from jax.experimental.pallas import tpu_sc as plsc

D = 64
T = 256
B = 1024
N = B * T
NC = 2
NS = 16
NW = NC * NS
BPW = N // NW
C = 512
NCHUNK = BPW // C
SREP = C // T


def _body(x_hbm, tok_hbm, pos_hbm, out_hbm, idx_v, rows0, rows1, pos_v,
          gsem, osem):
    wid = lax.axis_index("s") * NC + lax.axis_index("c")
    base = wid * BPW
    rows = (rows0, rows1)
    pltpu.sync_copy(x_hbm.at[pl.ds(base, BPW)], idx_v)
    pltpu.sync_copy(pos_hbm, pos_v)

    def start_gather(c, b):
        pltpu.async_copy(tok_hbm.at[idx_v.at[pl.ds(c * C, C)]], rows[b],
                         gsem.at[b])

    def add_pos(b):
        rv = rows[b]

        def add_body(t, _):
            for j in range(D // 16):
                vp = pos_v[t, pl.ds(j * 16, 16)]
                for s in range(SREP):
                    rv[s * T + t, pl.ds(j * 16, 16)] += vp
            return 0

        lax.fori_loop(0, T, add_body, 0)

    start_gather(0, 0)
    for c in range(NCHUNK):
        b = c & 1
        pltpu.make_async_copy(tok_hbm.at[idx_v.at[pl.ds(c * C, C)]],
                              rows[b], gsem.at[b]).wait()
        if c + 1 < NCHUNK:
            if c >= 1:
                pltpu.make_async_copy(
                    rows[1 - b],
                    out_hbm.at[pl.ds(base + (c - 1) * C, C)],
                    osem.at[1 - b]).wait()
            start_gather(c + 1, 1 - b)
        add_pos(b)
        pltpu.async_copy(rows[b], out_hbm.at[pl.ds(base + c * C, C)],
                         osem.at[b])
    pltpu.make_async_copy(rows[(NCHUNK - 1) & 1],
                          out_hbm.at[pl.ds(base + (NCHUNK - 1) * C, C)],
                          osem.at[(NCHUNK - 1) & 1]).wait()


def kernel(x, token_table, pos_table):
    xf = x.reshape(N).astype(jnp.int32)
    run = pl.kernel(
        _body,
        out_type=jax.ShapeDtypeStruct((N, D), jnp.float32),
        mesh=plsc.VectorSubcoreMesh(core_axis_name="c", subcore_axis_name="s"),
        compiler_params=pltpu.CompilerParams(use_tc_tiling_on_sc=False),
        scratch_types=[
            pltpu.VMEM((BPW,), jnp.int32),
            pltpu.VMEM((C, D), jnp.float32),
            pltpu.VMEM((C, D), jnp.float32),
            pltpu.VMEM((T, D), jnp.float32),
            pltpu.SemaphoreType.DMA((2,)),
            pltpu.SemaphoreType.DMA((2,)),
        ],
    )
    out = run(xf, token_table, pos_table)
    return out.reshape(B, T, D)
